# Initial kernel scaffold; baseline (speedup 1.0000x reference)
import jax
import jax.numpy as jnp
from jax.experimental import pallas as pl


def kernel(chunk_features, segment_ids, phi_w0, phi_b0, phi_w1, phi_b1, rho_w0, rho_b0, rho_w1, rho_b1, rho_w2, rho_b2):
    raise NotImplementedError("write your pallas kernel here")


# speedup vs baseline: 3.3317x; 3.3317x over previous
import functools

import jax
import jax.numpy as jnp
from jax import lax
from jax.experimental import pallas as pl
from jax.experimental.pallas import tpu as pltpu

N_CHUNKS = 160000
N_CELLS = 10000
IN_DIM = 256
HID = 512
OUT_DIM = 256

T = 512
W = 128
NT = (N_CHUNKS + T - 1) // T
PAD_CHUNKS = NT * T
PAD_ID = N_CELLS
PAD_CELLS = 10240
CT = 1000


def _phi_seg_body(ids_smem, x_ref, ids_vec_ref, w0_ref, b0_ref, w1_ref,
                  b1_ref, agg_ref):
    i = pl.program_id(0)

    @pl.when(i == 0)
    def _init():
        agg_ref[...] = jnp.zeros_like(agg_ref)

    x = x_ref[...]
    h = jnp.dot(x, w0_ref[...], preferred_element_type=jnp.float32)
    h = jnp.maximum(h + b0_ref[...], 0.0)
    h = jnp.dot(h, w1_ref[...], preferred_element_type=jnp.float32)
    h = jnp.maximum(h + b1_ref[...], 0.0)

    ids_vec = ids_vec_ref[0]

    def cond(p):
        return p < T

    def body(p):
        base = ids_smem[0, 0, p]
        base8 = (base // 8) * 8
        local = ids_vec - base8
        rows = lax.broadcasted_iota(jnp.int32, (W, T), 0)
        onehot = (rows == local).astype(jnp.float32)
        partial = jnp.dot(onehot, h, preferred_element_type=jnp.float32)
        agg_ref[pl.ds(base8, W), :] += partial
        p_new = jnp.sum((ids_vec < base8 + W).astype(jnp.int32))
        return p_new

    lax.while_loop(cond, body, jnp.int32(0))


def _rho_body(a_ref, w0_ref, b0_ref, w1_ref, b1_ref, w2_ref, b2_ref, o_ref):
    r = jnp.dot(a_ref[...], w0_ref[...], preferred_element_type=jnp.float32)
    r = jnp.maximum(r + b0_ref[...], 0.0)
    r = jnp.dot(r, w1_ref[...], preferred_element_type=jnp.float32)
    r = jnp.maximum(r + b1_ref[...], 0.0)
    o_ref[...] = jnp.dot(r, w2_ref[...],
                         preferred_element_type=jnp.float32) + b2_ref[...]


def kernel(chunk_features, segment_ids, phi_w0, phi_b0, phi_w1, phi_b1,
           rho_w0, rho_b0, rho_w1, rho_b1, rho_w2, rho_b2):
    ids = segment_ids.astype(jnp.int32)
    pad = PAD_CHUNKS - N_CHUNKS
    x = jnp.concatenate(
        [chunk_features,
         jnp.zeros((pad, IN_DIM), jnp.float32)], axis=0)
    ids = jnp.concatenate([ids, jnp.full((pad,), PAD_ID, jnp.int32)])
    ids3 = ids.reshape(NT, 1, T)

    agg = pl.pallas_call(
        _phi_seg_body,
        grid=(NT,),
        in_specs=[
            pl.BlockSpec((1, 1, T), lambda i: (i, 0, 0),
                         memory_space=pltpu.SMEM),
            pl.BlockSpec((T, IN_DIM), lambda i: (i, 0)),
            pl.BlockSpec((1, 1, T), lambda i: (i, 0, 0)),
            pl.BlockSpec((IN_DIM, HID), lambda i: (0, 0)),
            pl.BlockSpec((1, HID), lambda i: (0, 0)),
            pl.BlockSpec((HID, HID), lambda i: (0, 0)),
            pl.BlockSpec((1, HID), lambda i: (0, 0)),
        ],
        out_specs=pl.BlockSpec((PAD_CELLS, HID), lambda i: (0, 0)),
        out_shape=jax.ShapeDtypeStruct((PAD_CELLS, HID), jnp.float32),
        compiler_params=pltpu.CompilerParams(
            dimension_semantics=("arbitrary",)),
    )(ids3, x, ids3, phi_w0, phi_b0.reshape(1, HID), phi_w1,
      phi_b1.reshape(1, HID))

    agg = agg[:N_CELLS]

    out = pl.pallas_call(
        _rho_body,
        grid=(N_CELLS // CT,),
        in_specs=[
            pl.BlockSpec((CT, HID), lambda i: (i, 0)),
            pl.BlockSpec((HID, HID), lambda i: (0, 0)),
            pl.BlockSpec((1, HID), lambda i: (0, 0)),
            pl.BlockSpec((HID, HID), lambda i: (0, 0)),
            pl.BlockSpec((1, HID), lambda i: (0, 0)),
            pl.BlockSpec((HID, OUT_DIM), lambda i: (0, 0)),
            pl.BlockSpec((1, OUT_DIM), lambda i: (0, 0)),
        ],
        out_specs=pl.BlockSpec((CT, OUT_DIM), lambda i: (i, 0)),
        out_shape=jax.ShapeDtypeStruct((N_CELLS, OUT_DIM), jnp.float32),
    )(agg, rho_w0, rho_b0.reshape(1, HID), rho_w1, rho_b1.reshape(1, HID),
      rho_w2, rho_b2.reshape(1, OUT_DIM))
    return out
